# baseline (speedup 1.0000x reference)
import functools

import jax
import jax.numpy as jnp
from jax.experimental import pallas as pl
from jax.experimental.pallas import tpu as pltpu

_N = 8192
_D = 4096
_OUT = 4096
_THRESHOLD = 0.5

_BR = 512
_BW = 256
_BM = 512
_BN = 2048


def _stats_kernel(x_ref, w_ref, energy_ref, rowmean_ref, max_ref, wbf_ref,
                  xbf_ref):
    i = pl.program_id(0)
    xb = x_ref[...]
    absmean = jnp.mean(jnp.abs(xb), axis=1, keepdims=True)
    energy_ref[...] = absmean
    rowmean_ref[...] = jnp.mean(xb, axis=1, keepdims=True)
    wbf_ref[...] = w_ref[...].astype(jnp.bfloat16)
    xbf_ref[...] = xb.astype(jnp.bfloat16)
    bmax = jnp.max(absmean, axis=None, keepdims=True)

    @pl.when(i == 0)
    def _():
        max_ref[...] = bmax

    @pl.when(i > 0)
    def _():
        max_ref[...] = jnp.maximum(max_ref[...], bmax)


def _mm_kernel(x_ref, w_ref, b_ref, energy_ref, max_ref, rowmean_ref, o_ref):
    acc = jax.lax.dot_general(
        x_ref[...], w_ref[...],
        dimension_numbers=(((1,), (1,)), ((), ())),
        preferred_element_type=jnp.float32,
    )
    res = acc + b_ref[...]
    thresh = _THRESHOLD * (max_ref[...] + 1e-8)
    active = energy_ref[...] >= thresh
    o_ref[...] = jnp.where(active, res, rowmean_ref[...])


@jax.jit
def kernel(x, W, b):
    n, d = x.shape
    out_f = W.shape[0]

    energy, rowmean, emax, w_bf, x_bf = pl.pallas_call(
        _stats_kernel,
        grid=(n // _BR,),
        in_specs=[
            pl.BlockSpec((_BR, d), lambda i: (i, 0)),
            pl.BlockSpec((_BW, d), lambda i: (i, 0)),
        ],
        out_specs=[
            pl.BlockSpec((_BR, 1), lambda i: (i, 0)),
            pl.BlockSpec((_BR, 1), lambda i: (i, 0)),
            pl.BlockSpec((1, 1), lambda i: (0, 0)),
            pl.BlockSpec((_BW, d), lambda i: (i, 0)),
            pl.BlockSpec((_BR, d), lambda i: (i, 0)),
        ],
        out_shape=[
            jax.ShapeDtypeStruct((n, 1), jnp.float32),
            jax.ShapeDtypeStruct((n, 1), jnp.float32),
            jax.ShapeDtypeStruct((1, 1), jnp.float32),
            jax.ShapeDtypeStruct((out_f, d), jnp.bfloat16),
            jax.ShapeDtypeStruct((n, d), jnp.bfloat16),
        ],
    )(x, W)

    b2 = b.reshape(1, out_f)

    out = pl.pallas_call(
        _mm_kernel,
        grid=(out_f // _BN, n // _BM),
        in_specs=[
            pl.BlockSpec((_BM, d), lambda j, i: (i, 0)),
            pl.BlockSpec((_BN, d), lambda j, i: (j, 0)),
            pl.BlockSpec((1, _BN), lambda j, i: (0, j)),
            pl.BlockSpec((_BM, 1), lambda j, i: (i, 0)),
            pl.BlockSpec((1, 1), lambda j, i: (0, 0)),
            pl.BlockSpec((_BM, 1), lambda j, i: (i, 0)),
        ],
        out_specs=pl.BlockSpec((_BM, _BN), lambda j, i: (i, j)),
        out_shape=jax.ShapeDtypeStruct((n, out_f), jnp.float32),
        compiler_params=pltpu.CompilerParams(
            dimension_semantics=("arbitrary", "arbitrary"),
        ),
    )(x_bf, w_bf, b2, energy, emax, rowmean)
    return out


# speedup vs baseline: 1.0355x; 1.0355x over previous
import functools

import jax
import jax.numpy as jnp
from jax import lax
from jax.experimental import pallas as pl
from jax.experimental.pallas import tpu as pltpu
from jax.experimental.pallas import tpu_sc as plsc

_N = 8192
_D = 4096
_OUT = 4096
_THRESHOLD = 0.5

_BM = 512
_BN = 2048
_WCH = 128

_NC = 2
_NS = 16
_NW = _NC * _NS
_RPW = _N // _NW
_L = 16


def _stage_w(w_hbm_ref, wbf_ref, wc0_ref, wc1_ref, sem0, sem1, row0):
    nch = _BN // _WCH
    bufs = (wc0_ref, wc1_ref)
    sems = (sem0, sem1)
    copies = [
        pltpu.make_async_copy(
            w_hbm_ref.at[pl.ds(row0 + c * _WCH, _WCH), :], bufs[c % 2],
            sems[c % 2])
        for c in range(nch)
    ]
    copies[0].start()
    for c in range(1, nch):
        copies[c].start()
        copies[c - 1].wait()
        wbf_ref[pl.ds((c - 1) * _WCH, _WCH), :] = (
            bufs[(c - 1) % 2][...].astype(jnp.bfloat16))
    copies[nch - 1].wait()
    wbf_ref[pl.ds((nch - 1) * _WCH, _WCH), :] = (
        bufs[(nch - 1) % 2][...].astype(jnp.bfloat16))


def _k1_kernel(x_ref, w_hbm_ref, b_ref, o_ref, energy_ref, rowmean_ref,
               max_ref, wbf_ref, wc0_ref, wc1_ref, sem0, sem1):
    i = pl.program_id(0)

    @pl.when(i == 0)
    def _():
        _stage_w(w_hbm_ref, wbf_ref, wc0_ref, wc1_ref, sem0, sem1, 0)

    xb = x_ref[...]
    absmean = jnp.mean(jnp.abs(xb), axis=1, keepdims=True)
    energy_ref[...] = absmean
    rowmean_ref[...] = jnp.mean(xb, axis=1, keepdims=True)
    bmax = jnp.max(absmean)

    @pl.when(i == 0)
    def _():
        max_ref[...] = jnp.full((1, 128), bmax, jnp.float32)

    @pl.when(i > 0)
    def _():
        max_ref[...] = jnp.maximum(max_ref[...], bmax)

    acc = jax.lax.dot_general(
        xb.astype(jnp.bfloat16), wbf_ref[...],
        dimension_numbers=(((1,), (1,)), ((), ())),
        preferred_element_type=jnp.float32,
    )
    o_ref[...] = acc + b_ref[...]


def _k2_kernel(o_alias_ref, x_ref, w_hbm_ref, b_ref, energy_ref, max_ref,
               rowmean_ref, o_ref, wbf_ref, wc0_ref, wc1_ref, sem0, sem1):
    i = pl.program_id(0)

    @pl.when(i == 0)
    def _():
        _stage_w(w_hbm_ref, wbf_ref, wc0_ref, wc1_ref, sem0, sem1, _BN)

    acc = jax.lax.dot_general(
        x_ref[...].astype(jnp.bfloat16), wbf_ref[...],
        dimension_numbers=(((1,), (1,)), ((), ())),
        preferred_element_type=jnp.float32,
    )
    res = acc + b_ref[...]
    thresh = _THRESHOLD * (max_ref[...][:, 0:1] + 1e-8)
    active = energy_ref[...] >= thresh
    o_ref[...] = jnp.where(active, res, rowmean_ref[...])


def _patch_body(out_hbm, energy_hbm, rowmean_hbm, emax_hbm,
                e_all, m_v, t_v, fill_v):
    wid = lax.axis_index("s") * _NC + lax.axis_index("c")
    base = wid * _RPW
    pltpu.sync_copy(energy_hbm, e_all)
    pltpu.sync_copy(rowmean_hbm.at[pl.ds(base, _RPW)], m_v)
    pltpu.sync_copy(emax_hbm.at[pl.ds(0, _L)], t_v)
    threshv = _THRESHOLD * (t_v[...] + 1e-8)
    iot = lax.iota(jnp.int32, _L)

    def chunk_body(ci, _):
        e = e_all[pl.ds(base + ci * _L, _L)]
        inact = e < threshv
        cnt = plsc.all_reduce_population_count(inact)
        cnts = lax.reduce_max(cnt, axes=(0,))

        @pl.when(cnts > 0)
        def _():
            mchunk = m_v[pl.ds(ci * _L, _L)]
            inacti = jnp.where(inact, 1, 0)

            def row_body(r, _):
                flag = lax.reduce_max(
                    jnp.where(iot == r, inacti, 0), axes=(0,))

                @pl.when(flag > 0)
                def _():
                    val = lax.reduce_max(
                        jnp.where(iot == r, mchunk,
                                  jnp.full((_L,), -jnp.inf, jnp.float32)),
                        axes=(0,))
                    valv = jnp.full((_L,), val, jnp.float32)

                    def fill(k, _):
                        fill_v[pl.ds(k * _L, _L)] = valv
                        return 0

                    lax.fori_loop(0, _BN // _L, fill, 0)
                    row = base + ci * _L + r
                    pltpu.sync_copy(fill_v,
                                    out_hbm.at[row, pl.ds(0, _BN)])

                return 0

            lax.fori_loop(0, _L, row_body, 0)

        return 0

    lax.fori_loop(0, _RPW // _L, chunk_body, 0)


@jax.jit
def kernel(x, W, b):
    n, d = x.shape
    out_f = W.shape[0]

    b2 = b.reshape(1, out_f)

    mm_scratch = [
        pltpu.VMEM((_BN, _D), jnp.bfloat16),
        pltpu.VMEM((_WCH, _D), jnp.float32),
        pltpu.VMEM((_WCH, _D), jnp.float32),
        pltpu.SemaphoreType.DMA,
        pltpu.SemaphoreType.DMA,
    ]

    out1, energy, rowmean, emax = pl.pallas_call(
        _k1_kernel,
        grid=(n // _BM,),
        in_specs=[
            pl.BlockSpec((_BM, d), lambda i: (i, 0)),
            pl.BlockSpec(memory_space=pl.ANY),
            pl.BlockSpec((1, _BN), lambda i: (0, 0)),
        ],
        out_specs=[
            pl.BlockSpec((_BM, _BN), lambda i: (i, 0)),
            pl.BlockSpec((_BM, 1), lambda i: (i, 0)),
            pl.BlockSpec((_BM, 1), lambda i: (i, 0)),
            pl.BlockSpec((1, 128), lambda i: (0, 0)),
        ],
        out_shape=[
            jax.ShapeDtypeStruct((n, out_f), jnp.float32),
            jax.ShapeDtypeStruct((n, 1), jnp.float32),
            jax.ShapeDtypeStruct((n, 1), jnp.float32),
            jax.ShapeDtypeStruct((1, 128), jnp.float32),
        ],
        scratch_shapes=mm_scratch,
        compiler_params=pltpu.CompilerParams(
            dimension_semantics=("arbitrary",),
        ),
    )(x, W, b2)

    out2 = pl.pallas_call(
        _k2_kernel,
        grid=(n // _BM,),
        in_specs=[
            pl.BlockSpec(memory_space=pl.ANY),
            pl.BlockSpec((_BM, d), lambda i: (i, 0)),
            pl.BlockSpec(memory_space=pl.ANY),
            pl.BlockSpec((1, _BN), lambda i: (0, 1)),
            pl.BlockSpec((_BM, 1), lambda i: (i, 0)),
            pl.BlockSpec((1, 128), lambda i: (0, 0)),
            pl.BlockSpec((_BM, 1), lambda i: (i, 0)),
        ],
        out_specs=pl.BlockSpec((_BM, _BN), lambda i: (i, 1)),
        out_shape=jax.ShapeDtypeStruct((n, out_f), jnp.float32),
        input_output_aliases={0: 0},
        scratch_shapes=mm_scratch,
        compiler_params=pltpu.CompilerParams(
            dimension_semantics=("arbitrary",),
        ),
    )(out1, x, W, b2, energy, emax, rowmean)

    patch = pl.kernel(
        _patch_body,
        out_type=(),
        mesh=plsc.VectorSubcoreMesh(core_axis_name="c", subcore_axis_name="s"),
        scratch_types=[
            pltpu.VMEM((n,), jnp.float32),
            pltpu.VMEM((_RPW,), jnp.float32),
            pltpu.VMEM((_L,), jnp.float32),
            pltpu.VMEM((_BN,), jnp.float32),
        ],
        compiler_params=pltpu.CompilerParams(needs_layout_passes=False),
    )

    out_ref = jax.new_ref(out2)
    patch(out_ref, energy.reshape(n), rowmean.reshape(n), emax.reshape(128))
    return out_ref[...]


# speedup vs baseline: 1.0623x; 1.0259x over previous
import functools

import jax
import jax.numpy as jnp
from jax import lax
from jax.experimental import pallas as pl
from jax.experimental.pallas import tpu as pltpu
from jax.experimental.pallas import tpu_sc as plsc

_N = 8192
_D = 4096
_OUT = 4096
_THRESHOLD = 0.5

_BM = 256

_NC = 2
_NS = 16
_NW = _NC * _NS
_RPW = _N // _NW
_L = 16


_WCH = 128


def _mm_kernel(x_ref, w_hbm_ref, b_ref, o_ref, energy_ref, rowmean_ref,
               max_ref, wbf_ref, wc0_ref, wc1_ref, sem0, sem1):
    i = pl.program_id(0)

    @pl.when(i == 0)
    def _():
        nch = _OUT // _WCH
        bufs = (wc0_ref, wc1_ref)
        sems = (sem0, sem1)
        copies = [
            pltpu.make_async_copy(
                w_hbm_ref.at[pl.ds(c * _WCH, _WCH), :], bufs[c % 2],
                sems[c % 2])
            for c in range(nch)
        ]
        copies[0].start()
        for c in range(1, nch):
            copies[c].start()
            copies[c - 1].wait()
            wbf_ref[pl.ds((c - 1) * _WCH, _WCH), :] = (
                bufs[(c - 1) % 2][...].astype(jnp.bfloat16))
        copies[nch - 1].wait()
        wbf_ref[pl.ds((nch - 1) * _WCH, _WCH), :] = (
            bufs[(nch - 1) % 2][...].astype(jnp.bfloat16))
    xb = x_ref[...]
    absmean = jnp.mean(jnp.abs(xb), axis=1, keepdims=True)
    energy_ref[...] = absmean
    rowmean_ref[...] = jnp.mean(xb, axis=1, keepdims=True)
    bmax = jnp.max(absmean)

    @pl.when(i == 0)
    def _():
        max_ref[...] = jnp.full((1, 128), bmax, jnp.float32)

    @pl.when(i > 0)
    def _():
        max_ref[...] = jnp.maximum(max_ref[...], bmax)

    acc = jax.lax.dot_general(
        xb.astype(jnp.bfloat16), wbf_ref[...],
        dimension_numbers=(((1,), (1,)), ((), ())),
        preferred_element_type=jnp.float32,
    )
    o_ref[...] = acc + b_ref[...]


def _patch_body(out_hbm, energy_hbm, rowmean_hbm, emax_hbm,
                e_all, m_v, t_v, fill_v):
    wid = lax.axis_index("s") * _NC + lax.axis_index("c")
    base = wid * _RPW
    pltpu.sync_copy(energy_hbm, e_all)
    pltpu.sync_copy(rowmean_hbm.at[pl.ds(base, _RPW)], m_v)
    pltpu.sync_copy(emax_hbm.at[pl.ds(0, _L)], t_v)
    emax = lax.reduce_max(t_v[...], axes=(0,))
    thresh = _THRESHOLD * (emax + 1e-8)
    threshv = jnp.full((_L,), thresh, jnp.float32)
    iot = lax.iota(jnp.int32, _L)

    def chunk_body(ci, _):
        e = e_all[pl.ds(base + ci * _L, _L)]
        inact = e < threshv
        cnt = plsc.all_reduce_population_count(inact)
        cnts = lax.reduce_max(cnt, axes=(0,))

        @pl.when(cnts > 0)
        def _():
            mchunk = m_v[pl.ds(ci * _L, _L)]
            inacti = jnp.where(inact, 1, 0)

            def row_body(r, _):
                flag = lax.reduce_max(
                    jnp.where(iot == r, inacti, 0), axes=(0,))

                @pl.when(flag > 0)
                def _():
                    val = lax.reduce_max(
                        jnp.where(iot == r, mchunk,
                                  jnp.full((_L,), -jnp.inf, jnp.float32)),
                        axes=(0,))
                    valv = jnp.full((_L,), val, jnp.float32)

                    def fill(k, _):
                        fill_v[pl.ds(k * _L, _L)] = valv
                        return 0

                    lax.fori_loop(0, _D // _L, fill, 0)
                    row = base + ci * _L + r
                    pltpu.sync_copy(fill_v, out_hbm.at[row])

                return 0

            lax.fori_loop(0, _L, row_body, 0)

        return 0

    lax.fori_loop(0, _RPW // _L, chunk_body, 0)


@jax.jit
def kernel(x, W, b):
    n, d = x.shape
    out_f = W.shape[0]

    b2 = b.reshape(1, out_f)

    out_mm, energy, rowmean, emax = pl.pallas_call(
        _mm_kernel,
        grid=(n // _BM,),
        in_specs=[
            pl.BlockSpec((_BM, d), lambda i: (i, 0)),
            pl.BlockSpec(memory_space=pl.ANY),
            pl.BlockSpec((1, out_f), lambda i: (0, 0)),
        ],
        out_specs=[
            pl.BlockSpec((_BM, out_f), lambda i: (i, 0)),
            pl.BlockSpec((_BM, 1), lambda i: (i, 0)),
            pl.BlockSpec((_BM, 1), lambda i: (i, 0)),
            pl.BlockSpec((1, 128), lambda i: (0, 0)),
        ],
        out_shape=[
            jax.ShapeDtypeStruct((n, out_f), jnp.float32),
            jax.ShapeDtypeStruct((n, 1), jnp.float32),
            jax.ShapeDtypeStruct((n, 1), jnp.float32),
            jax.ShapeDtypeStruct((1, 128), jnp.float32),
        ],
        scratch_shapes=[
            pltpu.VMEM((_OUT, _D), jnp.bfloat16),
            pltpu.VMEM((_WCH, _D), jnp.float32),
            pltpu.VMEM((_WCH, _D), jnp.float32),
            pltpu.SemaphoreType.DMA,
            pltpu.SemaphoreType.DMA,
        ],
        compiler_params=pltpu.CompilerParams(
            dimension_semantics=("arbitrary",),
        ),
    )(x, W, b2)

    patch = pl.kernel(
        _patch_body,
        out_type=(),
        mesh=plsc.VectorSubcoreMesh(core_axis_name="c", subcore_axis_name="s"),
        scratch_types=[
            pltpu.VMEM((n,), jnp.float32),
            pltpu.VMEM((_RPW,), jnp.float32),
            pltpu.VMEM((_L,), jnp.float32),
            pltpu.VMEM((d,), jnp.float32),
        ],
        compiler_params=pltpu.CompilerParams(needs_layout_passes=False),
    )

    out_ref = jax.new_ref(out_mm)
    patch(out_ref, energy.reshape(n), rowmean.reshape(n), emax.reshape(128))
    return out_ref[...]


# speedup vs baseline: 1.0632x; 1.0009x over previous
import jax
import jax.numpy as jnp
from jax import lax
from jax.experimental import pallas as pl
from jax.experimental.pallas import tpu as pltpu
from jax.experimental.pallas import tpu_sc as plsc

_N = 8192
_D = 4096
_OUT = 4096
_THRESHOLD = 0.5

_BM = 256

_NC = 2
_NS = 16
_NW = _NC * _NS
_RPW = _N // _NW
_L = 16


_WCH = 128


def _mm_kernel(x_ref, w_hbm_ref, b_ref, o_ref, energy_ref, rowmean_ref,
               max_ref, wbf_ref, wc0_ref, wc1_ref, sem0, sem1):
    i = pl.program_id(0)

    @pl.when(i == 0)
    def _():
        nch = _OUT // _WCH
        bufs = (wc0_ref, wc1_ref)
        sems = (sem0, sem1)
        copies = [
            pltpu.make_async_copy(
                w_hbm_ref.at[pl.ds(c * _WCH, _WCH), :], bufs[c % 2],
                sems[c % 2])
            for c in range(nch)
        ]
        copies[0].start()
        for c in range(1, nch):
            copies[c].start()
            copies[c - 1].wait()
            wbf_ref[pl.ds((c - 1) * _WCH, _WCH), :] = (
                bufs[(c - 1) % 2][...].astype(jnp.bfloat16))
        copies[nch - 1].wait()
        wbf_ref[pl.ds((nch - 1) * _WCH, _WCH), :] = (
            bufs[(nch - 1) % 2][...].astype(jnp.bfloat16))
    xb = x_ref[...]
    absmean = jnp.mean(jnp.abs(xb), axis=1, keepdims=True)
    energy_ref[...] = absmean
    rowmean_ref[...] = jnp.mean(xb, axis=1, keepdims=True)
    bmax = jnp.max(absmean)

    @pl.when(i == 0)
    def _():
        max_ref[...] = jnp.full((1, 128), bmax, jnp.float32)

    @pl.when(i > 0)
    def _():
        max_ref[...] = jnp.maximum(max_ref[...], bmax)

    acc = jax.lax.dot_general(
        xb.astype(jnp.bfloat16), wbf_ref[...],
        dimension_numbers=(((1,), (1,)), ((), ())),
        preferred_element_type=jnp.float32,
    )
    o_ref[...] = acc + b_ref[...]


def _patch_body(out_hbm, energy_hbm, rowmean_hbm, emax_hbm,
                e_all, m_v, t_v, fill_v):
    wid = lax.axis_index("s") * _NC + lax.axis_index("c")
    base = wid * _RPW
    pltpu.sync_copy(energy_hbm, e_all)
    pltpu.sync_copy(rowmean_hbm.at[pl.ds(base, _RPW)], m_v)
    pltpu.sync_copy(emax_hbm.at[pl.ds(0, _L)], t_v)
    emax = lax.reduce_max(t_v[...], axes=(0,))
    thresh = _THRESHOLD * (emax + 1e-8)
    threshv = jnp.full((_L,), thresh, jnp.float32)
    iot = lax.iota(jnp.int32, _L)

    def chunk_body(ci, _):
        e = e_all[pl.ds(base + ci * _L, _L)]
        inact = e < threshv
        cnt = plsc.all_reduce_population_count(inact)
        cnts = lax.reduce_max(cnt, axes=(0,))

        @pl.when(cnts > 0)
        def _():
            mchunk = m_v[pl.ds(ci * _L, _L)]
            inacti = jnp.where(inact, 1, 0)

            def row_body(r, _):
                flag = lax.reduce_max(
                    jnp.where(iot == r, inacti, 0), axes=(0,))

                @pl.when(flag > 0)
                def _():
                    val = lax.reduce_max(
                        jnp.where(iot == r, mchunk,
                                  jnp.full((_L,), -jnp.inf, jnp.float32)),
                        axes=(0,))
                    valv = jnp.full((_L,), val, jnp.float32)

                    def fill(k, _):
                        fill_v[pl.ds(k * _L, _L)] = valv
                        return 0

                    lax.fori_loop(0, _D // _L, fill, 0)
                    row = base + ci * _L + r
                    pltpu.sync_copy(fill_v, out_hbm.at[row])

                return 0

            lax.fori_loop(0, _L, row_body, 0)

        return 0

    lax.fori_loop(0, _RPW // _L, chunk_body, 0)


@jax.jit
def kernel(x, W, b):
    n, d = x.shape
    out_f = W.shape[0]

    b2 = b.reshape(1, out_f)

    out_mm, energy, rowmean, emax = pl.pallas_call(
        _mm_kernel,
        grid=(n // _BM,),
        in_specs=[
            pl.BlockSpec((_BM, d), lambda i: (i, 0)),
            pl.BlockSpec(memory_space=pl.ANY),
            pl.BlockSpec((1, out_f), lambda i: (0, 0)),
        ],
        out_specs=[
            pl.BlockSpec((_BM, out_f), lambda i: (i, 0)),
            pl.BlockSpec((_BM, 1), lambda i: (i, 0)),
            pl.BlockSpec((_BM, 1), lambda i: (i, 0)),
            pl.BlockSpec((1, 128), lambda i: (0, 0)),
        ],
        out_shape=[
            jax.ShapeDtypeStruct((n, out_f), jnp.float32),
            jax.ShapeDtypeStruct((n, 1), jnp.float32),
            jax.ShapeDtypeStruct((n, 1), jnp.float32),
            jax.ShapeDtypeStruct((1, 128), jnp.float32),
        ],
        scratch_shapes=[
            pltpu.VMEM((_OUT, _D), jnp.bfloat16),
            pltpu.VMEM((_WCH, _D), jnp.float32),
            pltpu.VMEM((_WCH, _D), jnp.float32),
            pltpu.SemaphoreType.DMA,
            pltpu.SemaphoreType.DMA,
        ],
        compiler_params=pltpu.CompilerParams(
            dimension_semantics=("arbitrary",),
        ),
    )(x, W, b2)

    patch = pl.kernel(
        _patch_body,
        out_type=(),
        mesh=plsc.VectorSubcoreMesh(core_axis_name="c", subcore_axis_name="s"),
        scratch_types=[
            pltpu.VMEM((n,), jnp.float32),
            pltpu.VMEM((_RPW,), jnp.float32),
            pltpu.VMEM((_L,), jnp.float32),
            pltpu.VMEM((d,), jnp.float32),
        ],
        compiler_params=pltpu.CompilerParams(needs_layout_passes=False),
    )

    out_ref = jax.new_ref(out_mm)
    patch(out_ref, energy.reshape(n), rowmean.reshape(n), emax.reshape(128))
    return out_ref[...]
